# baseline (speedup 1.0000x reference)
import functools

import jax
import jax.numpy as jnp
from jax import lax
from jax.experimental import pallas as pl
from jax.experimental.pallas import tpu as pltpu
from jax.experimental.pallas import tpu_sc as plsc

N = 10000
E = 320000
D = 128
DF = 4
STR = 5
NT = 32
NCORE = 2
SB = 128
SEG = 25
NS = E // SB // SEG

R = 200
GRID = N // R


def _sc_aggregate(xp, src2d, dst2d):
    mesh = plsc.VectorSubcoreMesh(core_axis_name="c", subcore_axis_name="s")

    @functools.partial(
        pl.kernel,
        out_type=[
            jax.ShapeDtypeStruct((NT, N * STR), jnp.float32),
            jax.ShapeDtypeStruct((N,), jnp.float32),
        ],
        mesh=mesh,
        compiler_params=pltpu.CompilerParams(use_tc_tiling_on_sc=False,
                                             needs_layout_passes=False),
        scratch_types=[
            pltpu.VMEM((N * STR,), jnp.float32),
            pltpu.VMEM((N * STR,), jnp.float32),
            pltpu.VMEM((N,), jnp.float32),
            pltpu.VMEM((2, SEG, SB), jnp.int32),
            pltpu.VMEM((2, SEG, SB), jnp.int32),
            pltpu.SemaphoreType.DMA,
            pltpu.SemaphoreType.DMA,
            pltpu.SemaphoreType.DMA,
        ],
    )
    def body(xp_hbm, src_hbm, dst_hbm, agg_hbm, cnt_hbm,
             xtab_v, acc_v, cnt_v, src_v, dst_v, isem0, isem1, xsem):
        c = lax.axis_index("c")
        s = lax.axis_index("s")
        t = c * 16 + s
        iota = lax.iota(jnp.int32, 16)
        patq = iota >> 2
        colp = iota & 3
        zeros = jnp.zeros((16,), jnp.float32)
        ones = jnp.ones((16,), jnp.float32)
        isems = (isem0, isem1)

        pltpu.async_copy(xp_hbm.at[t], xtab_v, xsem)

        def fire_idx(seg, buf):
            r0 = seg * SEG
            pltpu.async_copy(src_hbm.at[pl.ds(r0, SEG)], src_v.at[buf],
                             isems[buf])
            pltpu.async_copy(dst_hbm.at[pl.ds(r0, SEG)], dst_v.at[buf],
                             isems[buf])

        fire_idx(0, 0)
        fire_idx(1, 1)

        def zacc(r, carry):
            acc_v[pl.ds(r * 16, 16)] = zeros
            return carry

        lax.fori_loop(0, N * STR // 16, zacc, None)

        def zcnt(r, carry):
            cnt_v[pl.ds(r * 16, 16)] = zeros
            return carry

        lax.fori_loop(0, N // 16, zcnt, None)

        pltpu.make_async_copy(xp_hbm.at[0], xtab_v, xsem).wait()

        def wait_idx(buf):
            pltpu.make_async_copy(src_hbm.at[pl.ds(0, SEG)], src_v.at[buf],
                                  isems[buf]).wait()
            pltpu.make_async_copy(dst_hbm.at[pl.ds(0, SEG)], dst_v.at[buf],
                                  isems[buf]).wait()

        def compute_segment(buf):
            def g_body(g, carry):
                jj = g >> 3
                m = g & 7
                e0 = m * 16
                src16 = src_v[buf, jj, pl.ds(e0, 16)]
                dst16 = dst_v[buf, jj, pl.ds(e0, 16)]
                dst5 = (dst16 << 2) + dst16
                plsc.addupdate_scatter(cnt_v, [dst16], ones)
                sidxs, datas = [], []
                for a in range(4):
                    pq = patq + 4 * a
                    ps = src16.at[pq].get(mode="promise_in_bounds")
                    pd = dst5.at[pq].get(mode="promise_in_bounds")
                    datas.append(plsc.load_gather(xtab_v, [ps + colp]))
                    sidxs.append(pd + colp)
                for a in range(4):
                    plsc.addupdate_scatter(acc_v, [sidxs[a]], datas[a])
                return carry

            lax.fori_loop(0, SEG * (SB // 16), g_body, None)

        def seg_pair(sp, carry):
            for b in (0, 1):
                seg = sp * 2 + b
                wait_idx(b)
                compute_segment(b)

                @pl.when(seg + 2 < NS)
                def _():
                    fire_idx(seg + 2, b)
            return carry

        lax.fori_loop(0, NS // 2, seg_pair, None)

        pltpu.sync_copy(acc_v, agg_hbm.at[t])

        @pl.when(t == 0)
        def _():
            pltpu.sync_copy(cnt_v, cnt_hbm)

    return body(xp, src2d, dst2d)


def _tc_combine(aggT, cnt, x, W_l, b_l, W_r):
    def body(agg_ref, cnt_ref, x_ref, wl_ref, wr_ref, b_ref, out_ref):
        a = agg_ref[...]
        cb = cnt_ref[0, 0]
        inv = 1.0 / jnp.maximum(cb, 1.0)
        mean = a * inv[:, None]
        dn = (((1,), (1,)), ((), ()))
        acc = lax.dot_general(mean, wl_ref[...], dn,
                              preferred_element_type=jnp.float32)
        acc = acc + lax.dot_general(x_ref[...], wr_ref[...], dn,
                                    preferred_element_type=jnp.float32)
        out_ref[...] = acc + b_ref[...]

    return pl.pallas_call(
        body,
        grid=(GRID,),
        in_specs=[
            pl.BlockSpec((R, D), lambda i: (i, 0)),
            pl.BlockSpec((1, 1, R), lambda i: (i, 0, 0)),
            pl.BlockSpec((R, D), lambda i: (i, 0)),
            pl.BlockSpec((D, D), lambda i: (0, 0)),
            pl.BlockSpec((D, D), lambda i: (0, 0)),
            pl.BlockSpec((1, D), lambda i: (0, 0)),
        ],
        out_specs=pl.BlockSpec((R, D), lambda i: (i, 0)),
        out_shape=jax.ShapeDtypeStruct((N, D), jnp.float32),
    )(aggT, cnt.reshape(GRID, 1, R), x, W_l, W_r, b_l)


def kernel(x, edge_index, W_l, b_l, W_r):
    ei = edge_index.astype(jnp.int32)
    src2d = (ei[0] * STR).reshape(E // SB, SB)
    dst2d = ei[1].reshape(E // SB, SB)
    x4 = x.reshape(N, NT, DF)
    xp = jnp.concatenate(
        [x4, jnp.zeros((N, NT, 1), jnp.float32)], axis=2
    ).transpose(1, 0, 2).reshape(NT, N * STR)
    agg, cnt = _sc_aggregate(xp, src2d, dst2d)
    aggT = (agg.reshape(NT, N, STR)[:, :, :DF]
            .transpose(1, 0, 2).reshape(N, D))
    return _tc_combine(aggT, cnt, x, W_l, b_l.reshape(1, D), W_r)


# speedup vs baseline: 1.5891x; 1.5891x over previous
import functools

import jax
import jax.numpy as jnp
from jax import lax
from jax.experimental import pallas as pl
from jax.experimental.pallas import tpu as pltpu
from jax.experimental.pallas import tpu_sc as plsc

N = 10000
E = 320000
D = 128
DF = 8
NSUB = 16
NCORE = 2
SB = 128
SEG = 25
CH = 5
NQ = SEG // CH
ROWS_PER_CORE = E // NCORE // SB
NS = ROWS_PER_CORE // SEG

R = 200
GRID = N // R


def _sc_aggregate(xg, src2d, dst2d):
    mesh = plsc.VectorSubcoreMesh(core_axis_name="c", subcore_axis_name="s")

    @functools.partial(
        pl.kernel,
        out_type=[
            jax.ShapeDtypeStruct((NCORE, N, D), jnp.float32),
            jax.ShapeDtypeStruct((NCORE, N), jnp.float32),
        ],
        mesh=mesh,
        compiler_params=pltpu.CompilerParams(use_tc_tiling_on_sc=False,
                                             needs_layout_passes=False),
        scratch_types=[
            pltpu.VMEM((N, DF), jnp.float32),
            pltpu.VMEM((N,), jnp.float32),
            pltpu.VMEM((2, SEG, SB), jnp.int32),
            pltpu.VMEM((2, SEG, SB), jnp.int32),
            pltpu.VMEM((2, CH, SB, DF), jnp.float32),
            pltpu.SemaphoreType.DMA,
            pltpu.SemaphoreType.DMA,
            pltpu.SemaphoreType.DMA,
            pltpu.SemaphoreType.DMA,
        ],
    )
    def body(x_hbm, src_hbm, dst_hbm, agg_hbm, cnt_hbm,
             acc_v, cnt_v, src_v, dst_v, rows_v, isem0, isem1, rsem0, rsem1):
        c = lax.axis_index("c")
        dc = lax.axis_index("s")
        iota = lax.iota(jnp.int32, 16)
        _PAT01 = iota >> 3
        _COLPAT = iota & 7
        _PATS = [_PAT01 + 2 * p for p in range(8)]
        zeros = jnp.zeros((16,), jnp.float32)
        ones = jnp.ones((16,), jnp.float32)
        isems = (isem0, isem1)
        rsems = (rsem0, rsem1)

        tbl = x_hbm.at[pl.ds(dc, N * NSUB - NSUB + 1)]

        def fire_idx(seg, buf):
            r0 = c * ROWS_PER_CORE + seg * SEG
            pltpu.async_copy(src_hbm.at[pl.ds(r0, SEG)], src_v.at[buf],
                             isems[buf])
            pltpu.async_copy(dst_hbm.at[pl.ds(r0, SEG)], dst_v.at[buf],
                             isems[buf])

        fire_idx(0, 0)
        fire_idx(1, 1)

        def zacc(r, carry):
            for u in range(8):
                row16 = (r * 16 + 2 * u) + _PAT01
                plsc.store_scatter(acc_v, [row16, _COLPAT], zeros)
            cnt_v[pl.ds(r * 16, 16)] = zeros
            return carry

        lax.fori_loop(0, N // 16, zacc, None)

        def wait_idx(buf):
            pltpu.make_async_copy(src_hbm.at[pl.ds(0, SEG)], src_v.at[buf],
                                  isems[buf]).wait()
            pltpu.make_async_copy(dst_hbm.at[pl.ds(0, SEG)], dst_v.at[buf],
                                  isems[buf]).wait()

        def fire_chunk(buf, q, rbuf):
            for i in range(CH):
                pltpu.async_copy(tbl.at[src_v.at[buf, q * CH + i]],
                                 rows_v.at[rbuf, i], rsems[rbuf])

        def compute(buf, q, rbuf):
            for i in range(CH):
                pltpu.make_async_copy(x_hbm.at[pl.ds(0, SB)],
                                      rows_v.at[rbuf, i], rsems[rbuf]).wait()

            def mg_body(mg, carry):
                i = mg >> 3
                m = mg & 7
                j = q * CH + i
                e0 = m * 16
                dst16 = dst_v[buf, j, pl.ds(e0, 16)]
                plsc.addupdate_scatter(cnt_v, [dst16], ones)
                idxs, datas = [], []
                for p in range(8):
                    idxs.append(dst16.at[_PATS[p]].get(
                        mode="promise_in_bounds"))
                    rpat = _PAT01 + (e0 + 2 * p)
                    datas.append(plsc.load_gather(rows_v.at[rbuf, i],
                                                  [rpat, _COLPAT]))
                for p in range(8):
                    plsc.addupdate_scatter(acc_v, [idxs[p], _COLPAT],
                                           datas[p])
                return carry

            lax.fori_loop(0, CH * (SB // 16), mg_body, None)

        wait_idx(0)
        fire_chunk(0, 0, 0)
        fire_chunk(0, 1, 1)

        def seg_pair(sp, carry):
            for b in (0, 1):
                seg = sp * 2 + b
                nb = 1 - b
                for q in range(NQ):
                    rb = (b + q) % 2
                    compute(b, q, rb)
                    t = q + 2
                    if t < NQ:
                        fire_chunk(b, t, (b + t) % 2)
                    elif t == NQ:
                        @pl.when(seg + 1 < NS)
                        def _():
                            wait_idx(nb)
                            fire_chunk(nb, 0, rb)
                    else:
                        @pl.when(seg + 1 < NS)
                        def _():
                            fire_chunk(nb, 1, rb)

                @pl.when(seg + 2 < NS)
                def _():
                    fire_idx(seg + 2, b)
            return carry

        lax.fori_loop(0, NS // 2, seg_pair, None)

        pltpu.sync_copy(acc_v, agg_hbm.at[c, :, pl.ds(dc * DF, DF)])

        @pl.when(dc == 0)
        def _():
            pltpu.sync_copy(cnt_v, cnt_hbm.at[c])

    return body(xg, src2d, dst2d)


def _tc_combine(agg, cnt, x, W_l, b_l, W_r):
    def body(agg_ref, cnt_ref, x_ref, wl_ref, wr_ref, b_ref, out_ref):
        a = agg_ref[0] + agg_ref[1]
        cb = cnt_ref[0, 0] + cnt_ref[0, 1]
        inv = 1.0 / jnp.maximum(cb, 1.0)
        mean = a * inv[:, None]
        dn = (((1,), (1,)), ((), ()))
        acc = lax.dot_general(mean, wl_ref[...], dn,
                              preferred_element_type=jnp.float32)
        acc = acc + lax.dot_general(x_ref[...], wr_ref[...], dn,
                                    preferred_element_type=jnp.float32)
        out_ref[...] = acc + b_ref[...]

    return pl.pallas_call(
        body,
        grid=(GRID,),
        in_specs=[
            pl.BlockSpec((NCORE, R, D), lambda i: (0, i, 0)),
            pl.BlockSpec((1, NCORE, R), lambda i: (i, 0, 0)),
            pl.BlockSpec((R, D), lambda i: (i, 0)),
            pl.BlockSpec((D, D), lambda i: (0, 0)),
            pl.BlockSpec((D, D), lambda i: (0, 0)),
            pl.BlockSpec((1, D), lambda i: (0, 0)),
        ],
        out_specs=pl.BlockSpec((R, D), lambda i: (i, 0)),
        out_shape=jax.ShapeDtypeStruct((N, D), jnp.float32),
    )(agg, cnt, x, W_l, W_r, b_l)


def kernel(x, edge_index, W_l, b_l, W_r):
    ei = edge_index.astype(jnp.int32)
    src2d = (ei[0] * NSUB).reshape(E // SB, SB)
    dst2d = ei[1].reshape(E // SB, SB)
    xg = x.reshape(N * NSUB, DF)
    agg, cnt = _sc_aggregate(xg, src2d, dst2d)
    cnt2 = cnt.reshape(NCORE, GRID, R).transpose(1, 0, 2)
    return _tc_combine(agg, cnt2, x, W_l, b_l.reshape(1, D), W_r)
